# baseline (speedup 1.0000x reference)
import functools

import jax
import jax.numpy as jnp
from jax import lax
from jax.experimental import pallas as pl
from jax.experimental.pallas import tpu as pltpu
from jax.experimental.pallas import tpu_sc as plsc

B = 16384
D = 64
NC = 2
NS = 16
L = 16
NW = NC * NS
BPW = B // NW
CH = 16
NCHK = BPW // CH

_mesh = plsc.VectorSubcoreMesh(core_axis_name="c", subcore_axis_name="s")


@functools.partial(
    pl.kernel,
    out_type=jax.ShapeDtypeStruct((B,), jnp.float32),
    mesh=_mesh,
    scratch_types=[
        pltpu.VMEM((BPW,), jnp.int32),
        pltpu.VMEM((BPW,), jnp.int32),
        pltpu.VMEM((BPW,), jnp.int32),
        pltpu.VMEM((CH, D), jnp.float32),
        pltpu.VMEM((CH, D), jnp.float32),
        pltpu.VMEM((CH, D), jnp.float32),
        pltpu.VMEM((CH, D), jnp.float32),
        pltpu.VMEM((CH, D), jnp.float32),
        pltpu.VMEM((CH, D), jnp.float32),
        pltpu.VMEM((BPW,), jnp.float32),
        pltpu.SemaphoreType.DMA,
        pltpu.SemaphoreType.DMA,
    ],
    compiler_params=pltpu.CompilerParams(needs_layout_passes=False),
)
def _bpr_kernel(uid_hbm, pid_hbm, nid_hbm, ut_hbm, it_hbm, out_hbm,
                idv_u, idv_p, idv_n, bu0, bu1, bp0, bp1, bn0, bn1,
                out_v, sem0, sem1):
    w = lax.axis_index("s") * NC + lax.axis_index("c")
    base = w * BPW

    pltpu.sync_copy(uid_hbm.at[pl.ds(base, BPW)], idv_u)
    pltpu.sync_copy(pid_hbm.at[pl.ds(base, BPW)], idv_p)
    pltpu.sync_copy(nid_hbm.at[pl.ds(base, BPW)], idv_n)

    bufs = ((bu0, bp0, bn0), (bu1, bp1, bn1))
    sems = (sem0, sem1)
    lane = lax.iota(jnp.int32, L)

    def fire_chunk(c, par):
        bu, bp, bn = bufs[par]
        sem = sems[par]
        sl = pl.ds(c * CH, CH)
        qv_u = idv_u[sl]
        qv_p = idv_p[sl]
        qv_n = idv_n[sl]
        for i in range(CH):
            q_u = qv_u[i]
            q_p = qv_p[i]
            q_n = qv_n[i]
            pltpu.make_async_copy(ut_hbm.at[q_u], bu.at[i], sem).start()
            pltpu.make_async_copy(it_hbm.at[q_p], bp.at[i], sem).start()
            pltpu.make_async_copy(it_hbm.at[q_n], bn.at[i], sem).start()

    def drain_chunk(par):
        bu, bp, bn = bufs[par]
        sem = sems[par]
        dummy = ut_hbm.at[pl.ds(0, CH)]
        pltpu.make_async_copy(dummy, bu, sem).wait()
        pltpu.make_async_copy(dummy, bp, sem).wait()
        pltpu.make_async_copy(dummy, bn, sem).wait()

    def compute_chunk(c, par):
        bu, bp, bn = bufs[par]

        def dbody(d, acc):
            dv = jnp.full((L,), d, jnp.int32)
            u = plsc.load_gather(bu, [lane, dv])
            p = plsc.load_gather(bp, [lane, dv])
            n = plsc.load_gather(bn, [lane, dv])
            return acc + u * (p - n)

        acc = lax.fori_loop(0, D, dbody, jnp.zeros((L,), jnp.float32),
                            unroll=8)
        out_v[pl.ds(c * CH, CH)] = acc

    fire_chunk(0, 0)
    fire_chunk(1, 1)

    def outer(t, carry):
        c0 = t * 2

        drain_chunk(0)
        compute_chunk(c0, 0)

        @pl.when(t < NCHK // 2 - 1)
        def _():
            fire_chunk(c0 + 2, 0)

        drain_chunk(1)
        compute_chunk(c0 + 1, 1)

        @pl.when(t < NCHK // 2 - 1)
        def _():
            fire_chunk(c0 + 3, 1)

        return carry

    lax.fori_loop(0, NCHK // 2, outer, 0)

    pltpu.sync_copy(out_v, out_hbm.at[pl.ds(base, BPW)])


def kernel(user_ids, pos_item_ids, neg_item_ids, user_table, item_table):
    uid = user_ids.astype(jnp.int32)
    pid = pos_item_ids.astype(jnp.int32)
    nid = neg_item_ids.astype(jnp.int32)
    return _bpr_kernel(uid, pid, nid, user_table, item_table)
